# baseline (speedup 1.0000x reference)
import jax
import jax.numpy as jnp
from jax.experimental import pallas as pl


def _affine_body(h_ref, g_ref, b_ref, o_ref):
    o_ref[...] = h_ref[...] * g_ref[...] + b_ref[...]


def kernel(x, edge_index, W_l0, b_l0, W_r0, gamma0, beta0, W_l1, b_l1, W_r1):
    N = x.shape[0]
    src = edge_index[0]
    dst = edge_index[1]

    msg = jnp.take(x, src, axis=0)
    agg = jax.ops.segment_max(msg, dst, num_segments=N)
    agg = jnp.where(jnp.isneginf(agg), 0.0, agg)
    h = agg @ W_l0 + b_l0 + x @ W_r0
    mu = jnp.mean(h, axis=0)
    var = jnp.var(h, axis=0)
    scale = gamma0 / jnp.sqrt(var + 1e-5)
    shift = beta0 - mu * scale
    h = pl.pallas_call(
        _affine_body,
        out_shape=jax.ShapeDtypeStruct(h.shape, h.dtype),
    )(h, jnp.broadcast_to(scale, h.shape), jnp.broadcast_to(shift, h.shape))
    h = jax.nn.relu(h)
    msg1 = jnp.take(h, src, axis=0)
    agg1 = jax.ops.segment_max(msg1, dst, num_segments=N)
    agg1 = jnp.where(jnp.isneginf(agg1), 0.0, agg1)
    return agg1 @ W_l1 + b_l1 + h @ W_r1


# speedup vs baseline: 2.1131x; 2.1131x over previous
import functools

import jax
import jax.numpy as jnp
from jax import lax
from jax.experimental import pallas as pl
from jax.experimental.pallas import tpu as pltpu
from jax.experimental.pallas import tpu_sc as plsc

NN = 10000
EE = 320000
DD = 128

NTILE = 32
RPT = 320
ROWS = RPT + 1
NPAD = NTILE * RPT

SCH = 2000
NCHUNK = EE // SCH
GRP = SCH // 16

FLUSH = 2048
ABUF = 4096 + 16
CAP = FLUSH * 158
PADM = 256

LCH = 2048
GE = 64

_SC_PARAMS = pltpu.CompilerParams(needs_layout_passes=False)


def _wid():
    return lax.axis_index("s") * 2 + lax.axis_index("c")


def _p1_body(src_hbm, dst_hbm, slist, dlist, counts,
             sbuf0, sbuf1, dbuf0, dbuf1, asrc, adst, cbuf, ss0, ss1, sd0, sd1):
    wid = _wid()
    lo = wid * RPT
    sbufs = (sbuf0, sbuf1)
    dbufs = (dbuf0, dbuf1)
    ssems = (ss0, ss1)
    dsems = (sd0, sd1)

    for b in (0, 1):
        pltpu.async_copy(src_hbm.at[pl.ds(b * SCH, SCH)], sbufs[b], ssems[b])
        pltpu.async_copy(dst_hbm.at[pl.ds(b * SCH, SCH)], dbufs[b], dsems[b])

    def outer(p, carry):
        aoff, hoff = carry
        for b in (0, 1):
            c = p * 2 + b
            pltpu.make_async_copy(
                src_hbm.at[pl.ds(0, SCH)], sbufs[b], ssems[b]).wait()
            pltpu.make_async_copy(
                dst_hbm.at[pl.ds(0, SCH)], dbufs[b], dsems[b]).wait()

            def inner(g, ch, _b=b):
                aoff, hoff = ch
                dvec = dbufs[_b][pl.ds(g * 16, 16)]
                svec = sbufs[_b][pl.ds(g * 16, 16)]
                dloc = dvec - lo
                mask = dloc.astype(jnp.uint32) < jnp.uint32(RPT)
                plsc.store_compressed(adst.at[pl.ds(aoff, 16)], dloc, mask=mask)
                plsc.store_compressed(asrc.at[pl.ds(aoff, 16)], svec, mask=mask)
                cntv = plsc.all_reduce_population_count(mask)
                cnt = cntv[0] if getattr(cntv, "ndim", 0) else cntv
                aoff = aoff + cnt
                do_flush = aoff >= FLUSH

                @pl.when(do_flush)
                def _():
                    off = pl.multiple_of(wid * CAP + hoff, FLUSH)
                    pltpu.sync_copy(asrc.at[pl.ds(0, FLUSH)],
                                    slist.at[pl.ds(off, FLUSH)])
                    pltpu.sync_copy(adst.at[pl.ds(0, FLUSH)],
                                    dlist.at[pl.ds(off, FLUSH)])
                    rs = asrc[pl.ds(FLUSH, 16)]
                    rd = adst[pl.ds(FLUSH, 16)]
                    asrc[pl.ds(0, 16)] = rs
                    adst[pl.ds(0, 16)] = rd

                aoff = jnp.where(do_flush, aoff - FLUSH, aoff)
                hoff = jnp.where(do_flush, hoff + FLUSH, hoff)
                return aoff, hoff

            aoff, hoff = lax.fori_loop(0, GRP, inner, (aoff, hoff))

            @pl.when(c + 2 < NCHUNK)
            def _(_b=b, _c=c):
                off = (_c + 2) * SCH
                pltpu.async_copy(src_hbm.at[pl.ds(off, SCH)], sbufs[_b],
                                 ssems[_b])
                pltpu.async_copy(dst_hbm.at[pl.ds(off, SCH)], dbufs[_b],
                                 dsems[_b])
        return aoff, hoff

    aoff, hoff = lax.fori_loop(0, NCHUNK // 2, outer, (0, 0))

    npad = (PADM - ((hoff + aoff) & (PADM - 1))) & (PADM - 1)
    padsrc = wid * RPT + lax.iota(jnp.int32, 16)
    padloc = jnp.full((16,), RPT, jnp.int32)

    def padloop(i, _):
        asrc[pl.ds(aoff + i * 16, 16)] = padsrc
        adst[pl.ds(aoff + i * 16, 16)] = padloc
        return 0

    lax.fori_loop(0, (npad + 15) >> 4, padloop, 0)
    aoff = aoff + npad

    for blk in (0, 1):
        off = pl.multiple_of(wid * CAP + hoff + blk * FLUSH, FLUSH)
        pltpu.sync_copy(asrc.at[pl.ds(blk * FLUSH, FLUSH)],
                        slist.at[pl.ds(off, FLUSH)])
        pltpu.sync_copy(adst.at[pl.ds(blk * FLUSH, FLUSH)],
                        dlist.at[pl.ds(off, FLUSH)])

    total = hoff + aoff
    cbuf[pl.ds(0, 16)] = jnp.full((16,), 0, jnp.int32) + total
    pltpu.sync_copy(cbuf, counts.at[pl.ds(wid * 16, 16)])


def _p1(src, dst):
    mesh = plsc.VectorSubcoreMesh(core_axis_name="c", subcore_axis_name="s")
    fn = pl.kernel(
        _p1_body,
        out_type=(
            jax.ShapeDtypeStruct((NTILE * CAP,), jnp.int32),
            jax.ShapeDtypeStruct((NTILE * CAP,), jnp.int32),
            jax.ShapeDtypeStruct((NTILE * 16,), jnp.int32),
        ),
        mesh=mesh,
        compiler_params=_SC_PARAMS,
        scratch_types=[
            pltpu.VMEM((SCH,), jnp.int32),
            pltpu.VMEM((SCH,), jnp.int32),
            pltpu.VMEM((SCH,), jnp.int32),
            pltpu.VMEM((SCH,), jnp.int32),
            pltpu.VMEM((ABUF,), jnp.int32),
            pltpu.VMEM((ABUF,), jnp.int32),
            pltpu.VMEM((16,), jnp.int32),
            pltpu.SemaphoreType.DMA,
            pltpu.SemaphoreType.DMA,
            pltpu.SemaphoreType.DMA,
            pltpu.SemaphoreType.DMA,
        ],
    )
    return fn(src, dst)


def _p2_body(tab_hbm, slist, dlist, counts, out_hbm,
             acc, sbuf, dbuf, cbuf, msg0, msg1, m0, m1):
    wid = _wid()
    msgs = (msg0, msg1)
    msems = (m0, m1)

    pltpu.sync_copy(counts.at[pl.ds(wid * 16, 16)], cbuf)
    total = cbuf[pl.ds(0, 16)][0]

    neg = jnp.full((16,), -jnp.inf, jnp.float32)

    def initloop(r, _):
        for ci in range(8):
            acc[r, pl.ds(ci * 16, 16)] = neg
        return 0

    lax.fori_loop(0, ROWS, initloop, 0)

    nch = (total + (LCH - 1)) >> 11

    def chunk_body(ci, _):
        base = pl.multiple_of(wid * CAP + ci * LCH, LCH)
        pltpu.sync_copy(slist.at[pl.ds(base, LCH)], sbuf)
        pltpu.sync_copy(dlist.at[pl.ds(base, LCH)], dbuf)
        gcount = jnp.minimum(total - ci * LCH, LCH) >> 6

        def group_body(g, _):
            pltpu.async_copy(tab_hbm.at[sbuf.at[pl.ds(g * GE, GE)]],
                             msg0, m0).wait()

            def sub(q, _):
                dvec = dbuf[pl.ds(g * GE + q * 16, 16)]
                for j in range(16):
                    d = dvec[j]
                    e = q * 16 + j
                    for cc in range(8):
                        cur = acc[d, pl.ds(cc * 16, 16)]
                        m = msg0[e, pl.ds(cc * 16, 16)]
                        acc[d, pl.ds(cc * 16, 16)] = jnp.maximum(cur, m)
                return 0

            lax.fori_loop(0, 4, sub, 0)
            return 0

        lax.fori_loop(0, gcount, group_body, 0)
        return 0

    lax.fori_loop(0, nch, chunk_body, 0)

    row0 = pl.multiple_of(wid * RPT, 8)
    pltpu.sync_copy(acc.at[pl.ds(0, RPT)], out_hbm.at[pl.ds(row0, RPT)])


def _p2(tab, slist, dlist, counts):
    mesh = plsc.VectorSubcoreMesh(core_axis_name="c", subcore_axis_name="s")
    fn = pl.kernel(
        _p2_body,
        out_type=jax.ShapeDtypeStruct((NPAD, DD), jnp.float32),
        mesh=mesh,
        compiler_params=_SC_PARAMS,
        scratch_types=[
            pltpu.VMEM((ROWS, DD), jnp.float32),
            pltpu.VMEM((LCH,), jnp.int32),
            pltpu.VMEM((LCH,), jnp.int32),
            pltpu.VMEM((16,), jnp.int32),
            pltpu.VMEM((GE, DD), jnp.float32),
            pltpu.VMEM((GE, DD), jnp.float32),
            pltpu.SemaphoreType.DMA,
            pltpu.SemaphoreType.DMA,
        ],
    )
    return fn(tab, slist, dlist, counts)


BN = 1000
GRID = NN // BN

_DOT = functools.partial(
    lax.dot_general,
    dimension_numbers=(((1,), (0,)), ((), ())),
    preferred_element_type=jnp.float32,
    precision=lax.Precision.HIGHEST,
)


def _sage_stats_body(a_ref, x_ref, wl_ref, wr_ref, b_ref,
                     h_ref, sum_ref, sq_ref):
    i = pl.program_id(0)
    a = a_ref[...]
    a = jnp.where(a == -jnp.inf, 0.0, a)
    h = _DOT(a, wl_ref[...]) + _DOT(x_ref[...], wr_ref[...]) + b_ref[...]
    h_ref[...] = h

    @pl.when(i == 0)
    def _():
        sum_ref[...] = jnp.zeros_like(sum_ref)
        sq_ref[...] = jnp.zeros_like(sq_ref)

    sum_ref[...] += jnp.sum(h, axis=0, keepdims=True)
    sq_ref[...] += jnp.sum(h * h, axis=0, keepdims=True)


def _sage_body(a_ref, x_ref, wl_ref, wr_ref, b_ref, h_ref):
    a = a_ref[...]
    a = jnp.where(a == -jnp.inf, 0.0, a)
    h_ref[...] = _DOT(a, wl_ref[...]) + _DOT(x_ref[...], wr_ref[...]) + b_ref[...]


def _bn_relu_body(h_ref, sum_ref, sq_ref, g_ref, be_ref, o_ref):
    mu = sum_ref[...] / NN
    var = sq_ref[...] / NN - mu * mu
    scale = g_ref[...] * lax.rsqrt(var + 1e-5)
    shift = be_ref[...] - mu * scale
    o_ref[...] = jnp.maximum(h_ref[...] * scale + shift, 0.0)


_ROWBLK = pl.BlockSpec((BN, DD), lambda i: (i, 0))
_FULLW = pl.BlockSpec((DD, DD), lambda i: (0, 0))
_VEC = pl.BlockSpec((1, DD), lambda i: (0, 0))


def _sage_stats(agg, x, wl, wr, b):
    return pl.pallas_call(
        _sage_stats_body,
        grid=(GRID,),
        in_specs=[_ROWBLK, _ROWBLK, _FULLW, _FULLW, _VEC],
        out_specs=[_ROWBLK, _VEC, _VEC],
        out_shape=[
            jax.ShapeDtypeStruct((NN, DD), jnp.float32),
            jax.ShapeDtypeStruct((1, DD), jnp.float32),
            jax.ShapeDtypeStruct((1, DD), jnp.float32),
        ],
    )(agg, x, wl, wr, b.reshape(1, DD))


def _sage(agg, x, wl, wr, b):
    return pl.pallas_call(
        _sage_body,
        grid=(GRID,),
        in_specs=[_ROWBLK, _ROWBLK, _FULLW, _FULLW, _VEC],
        out_specs=_ROWBLK,
        out_shape=jax.ShapeDtypeStruct((NN, DD), jnp.float32),
    )(agg, x, wl, wr, b.reshape(1, DD))


def _bn_relu(h, s, sq, gamma, beta):
    return pl.pallas_call(
        _bn_relu_body,
        grid=(GRID,),
        in_specs=[_ROWBLK, _VEC, _VEC, _VEC, _VEC],
        out_specs=_ROWBLK,
        out_shape=jax.ShapeDtypeStruct((NN, DD), jnp.float32),
    )(h, s, sq, gamma.reshape(1, DD), beta.reshape(1, DD))


def kernel(x, edge_index, W_l0, b_l0, W_r0, gamma0, beta0, W_l1, b_l1, W_r1):
    src = edge_index[0]
    dst = edge_index[1]
    slist, dlist, counts = _p1(src, dst)
    agg0 = _p2(x, slist, dlist, counts)[:NN]
    h_pre, s, sq = _sage_stats(agg0, x, W_l0, W_r0, b_l0)
    h = _bn_relu(h_pre, s, sq, gamma0, beta0)
    agg1 = _p2(h, slist, dlist, counts)[:NN]
    return _sage(agg1, h, W_l1, W_r1, b_l1)


# speedup vs baseline: 2.4918x; 1.1792x over previous
import functools

import jax
import jax.numpy as jnp
from jax import lax
from jax.experimental import pallas as pl
from jax.experimental.pallas import tpu as pltpu
from jax.experimental.pallas import tpu_sc as plsc

NN = 10000
EE = 320000
DD = 128

NTILE = 32
RPT = 320
ROWS = RPT + 1
NPAD = NTILE * RPT

SCH = 2000
NCHUNK = EE // SCH
GRP = SCH // 16

FLUSH = 2048
ABUF = 4096 + 16
CAP = FLUSH * 158
PADM = 256

LCH = 2048
GE = 64

_SC_PARAMS = pltpu.CompilerParams(needs_layout_passes=False)


def _wid():
    return lax.axis_index("s") * 2 + lax.axis_index("c")


def _p1_body(src_hbm, dst_hbm, slist, dlist, counts,
             sbuf0, sbuf1, dbuf0, dbuf1, asrc, adst, cbuf, ss0, ss1, sd0, sd1):
    wid = _wid()
    lo = wid * RPT
    sbufs = (sbuf0, sbuf1)
    dbufs = (dbuf0, dbuf1)
    ssems = (ss0, ss1)
    dsems = (sd0, sd1)

    for b in (0, 1):
        pltpu.async_copy(src_hbm.at[pl.ds(b * SCH, SCH)], sbufs[b], ssems[b])
        pltpu.async_copy(dst_hbm.at[pl.ds(b * SCH, SCH)], dbufs[b], dsems[b])

    def outer(p, carry):
        aoff, hoff = carry
        for b in (0, 1):
            c = p * 2 + b
            pltpu.make_async_copy(
                src_hbm.at[pl.ds(0, SCH)], sbufs[b], ssems[b]).wait()
            pltpu.make_async_copy(
                dst_hbm.at[pl.ds(0, SCH)], dbufs[b], dsems[b]).wait()

            def inner(g, ch, _b=b):
                aoff, hoff = ch
                dvec = dbufs[_b][pl.ds(g * 16, 16)]
                svec = sbufs[_b][pl.ds(g * 16, 16)]
                dloc = dvec - lo
                mask = dloc.astype(jnp.uint32) < jnp.uint32(RPT)
                plsc.store_compressed(adst.at[pl.ds(aoff, 16)], dloc, mask=mask)
                plsc.store_compressed(asrc.at[pl.ds(aoff, 16)], svec, mask=mask)
                cntv = plsc.all_reduce_population_count(mask)
                cnt = cntv[0] if getattr(cntv, "ndim", 0) else cntv
                aoff = aoff + cnt
                do_flush = aoff >= FLUSH

                @pl.when(do_flush)
                def _():
                    off = pl.multiple_of(wid * CAP + hoff, FLUSH)
                    pltpu.sync_copy(asrc.at[pl.ds(0, FLUSH)],
                                    slist.at[pl.ds(off, FLUSH)])
                    pltpu.sync_copy(adst.at[pl.ds(0, FLUSH)],
                                    dlist.at[pl.ds(off, FLUSH)])
                    rs = asrc[pl.ds(FLUSH, 16)]
                    rd = adst[pl.ds(FLUSH, 16)]
                    asrc[pl.ds(0, 16)] = rs
                    adst[pl.ds(0, 16)] = rd

                aoff = jnp.where(do_flush, aoff - FLUSH, aoff)
                hoff = jnp.where(do_flush, hoff + FLUSH, hoff)
                return aoff, hoff

            aoff, hoff = lax.fori_loop(0, GRP, inner, (aoff, hoff))

            @pl.when(c + 2 < NCHUNK)
            def _(_b=b, _c=c):
                off = (_c + 2) * SCH
                pltpu.async_copy(src_hbm.at[pl.ds(off, SCH)], sbufs[_b],
                                 ssems[_b])
                pltpu.async_copy(dst_hbm.at[pl.ds(off, SCH)], dbufs[_b],
                                 dsems[_b])
        return aoff, hoff

    aoff, hoff = lax.fori_loop(0, NCHUNK // 2, outer, (0, 0))

    npad = (PADM - ((hoff + aoff) & (PADM - 1))) & (PADM - 1)
    padsrc = wid * RPT + lax.iota(jnp.int32, 16)
    padloc = jnp.full((16,), RPT, jnp.int32)

    def padloop(i, _):
        asrc[pl.ds(aoff + i * 16, 16)] = padsrc
        adst[pl.ds(aoff + i * 16, 16)] = padloc
        return 0

    lax.fori_loop(0, (npad + 15) >> 4, padloop, 0)
    aoff = aoff + npad

    for blk in (0, 1):
        off = pl.multiple_of(wid * CAP + hoff + blk * FLUSH, FLUSH)
        pltpu.sync_copy(asrc.at[pl.ds(blk * FLUSH, FLUSH)],
                        slist.at[pl.ds(off, FLUSH)])
        pltpu.sync_copy(adst.at[pl.ds(blk * FLUSH, FLUSH)],
                        dlist.at[pl.ds(off, FLUSH)])

    total = hoff + aoff
    cbuf[pl.ds(0, 16)] = jnp.full((16,), 0, jnp.int32) + total
    pltpu.sync_copy(cbuf, counts.at[pl.ds(wid * 16, 16)])


def _p1(src, dst):
    mesh = plsc.VectorSubcoreMesh(core_axis_name="c", subcore_axis_name="s")
    fn = pl.kernel(
        _p1_body,
        out_type=(
            jax.ShapeDtypeStruct((NTILE * CAP,), jnp.int32),
            jax.ShapeDtypeStruct((NTILE * CAP,), jnp.int32),
            jax.ShapeDtypeStruct((NTILE * 16,), jnp.int32),
        ),
        mesh=mesh,
        compiler_params=_SC_PARAMS,
        scratch_types=[
            pltpu.VMEM((SCH,), jnp.int32),
            pltpu.VMEM((SCH,), jnp.int32),
            pltpu.VMEM((SCH,), jnp.int32),
            pltpu.VMEM((SCH,), jnp.int32),
            pltpu.VMEM((ABUF,), jnp.int32),
            pltpu.VMEM((ABUF,), jnp.int32),
            pltpu.VMEM((16,), jnp.int32),
            pltpu.SemaphoreType.DMA,
            pltpu.SemaphoreType.DMA,
            pltpu.SemaphoreType.DMA,
            pltpu.SemaphoreType.DMA,
        ],
    )
    return fn(src, dst)


def _p2_body(tab_hbm, slist, dlist, counts, out_hbm,
             acc, sbuf, dbuf, cbuf, msg0, msg1, msg2, msg3, m0, m1, m2, m3):
    wid = _wid()
    msgs = (msg0, msg1, msg2, msg3)
    msems = (m0, m1, m2, m3)

    pltpu.sync_copy(counts.at[pl.ds(wid * 16, 16)], cbuf)
    total = cbuf[pl.ds(0, 16)][0]

    neg = jnp.full((16,), -jnp.inf, jnp.float32)

    def initloop(r, _):
        for ci in range(8):
            acc[r, pl.ds(ci * 16, 16)] = neg
        return 0

    lax.fori_loop(0, ROWS, initloop, 0)

    nch = (total + (LCH - 1)) >> 11

    def chunk_body(ci, _):
        base = pl.multiple_of(wid * CAP + ci * LCH, LCH)
        pltpu.sync_copy(slist.at[pl.ds(base, LCH)], sbuf)
        pltpu.sync_copy(dlist.at[pl.ds(base, LCH)], dbuf)
        gcount = jnp.minimum(total - ci * LCH, LCH) >> 6

        for b in range(4):
            pltpu.async_copy(tab_hbm.at[sbuf.at[pl.ds(b * GE, GE)]],
                             msgs[b], msems[b])

        def quad_body(qd, _):
            for b in range(4):
                g = qd * 4 + b
                pltpu.make_async_copy(tab_hbm.at[pl.ds(0, GE)], msgs[b],
                                      msems[b]).wait()

                def sub(q, _, _b=b):
                    dvec = dbuf[pl.ds(g * GE + q * 16, 16)]
                    for j in range(16):
                        d = dvec[j]
                        e = q * 16 + j
                        for cc in range(8):
                            cur = acc[d, pl.ds(cc * 16, 16)]
                            m = msgs[_b][e, pl.ds(cc * 16, 16)]
                            acc[d, pl.ds(cc * 16, 16)] = jnp.maximum(cur, m)
                    return 0

                lax.fori_loop(0, 4, sub, 0)
                gg = jnp.minimum(g + 4, gcount - 1)
                pltpu.async_copy(tab_hbm.at[sbuf.at[pl.ds(gg * GE, GE)]],
                                 msgs[b], msems[b])
            return 0

        lax.fori_loop(0, gcount >> 2, quad_body, 0)
        for b in range(4):
            pltpu.make_async_copy(tab_hbm.at[pl.ds(0, GE)], msgs[b],
                                  msems[b]).wait()
        return 0

    lax.fori_loop(0, nch, chunk_body, 0)

    row0 = pl.multiple_of(wid * RPT, 8)
    pltpu.sync_copy(acc.at[pl.ds(0, RPT)], out_hbm.at[pl.ds(row0, RPT)])


def _p2(tab, slist, dlist, counts):
    mesh = plsc.VectorSubcoreMesh(core_axis_name="c", subcore_axis_name="s")
    fn = pl.kernel(
        _p2_body,
        out_type=jax.ShapeDtypeStruct((NPAD, DD), jnp.float32),
        mesh=mesh,
        compiler_params=_SC_PARAMS,
        scratch_types=[
            pltpu.VMEM((ROWS, DD), jnp.float32),
            pltpu.VMEM((LCH,), jnp.int32),
            pltpu.VMEM((LCH,), jnp.int32),
            pltpu.VMEM((16,), jnp.int32),
            pltpu.VMEM((GE, DD), jnp.float32),
            pltpu.VMEM((GE, DD), jnp.float32),
            pltpu.VMEM((GE, DD), jnp.float32),
            pltpu.VMEM((GE, DD), jnp.float32),
            pltpu.SemaphoreType.DMA,
            pltpu.SemaphoreType.DMA,
            pltpu.SemaphoreType.DMA,
            pltpu.SemaphoreType.DMA,
        ],
    )
    return fn(tab, slist, dlist, counts)


BN = 1000
GRID = NN // BN

_DOT = functools.partial(
    lax.dot_general,
    dimension_numbers=(((1,), (0,)), ((), ())),
    preferred_element_type=jnp.float32,
    precision=lax.Precision.HIGHEST,
)


def _sage_stats_body(a_ref, x_ref, wl_ref, wr_ref, b_ref,
                     h_ref, sum_ref, sq_ref):
    i = pl.program_id(0)
    a = a_ref[...]
    a = jnp.where(a == -jnp.inf, 0.0, a)
    h = _DOT(a, wl_ref[...]) + _DOT(x_ref[...], wr_ref[...]) + b_ref[...]
    h_ref[...] = h

    @pl.when(i == 0)
    def _():
        sum_ref[...] = jnp.zeros_like(sum_ref)
        sq_ref[...] = jnp.zeros_like(sq_ref)

    sum_ref[...] += jnp.sum(h, axis=0, keepdims=True)
    sq_ref[...] += jnp.sum(h * h, axis=0, keepdims=True)


def _sage_body(a_ref, x_ref, wl_ref, wr_ref, b_ref, h_ref):
    a = a_ref[...]
    a = jnp.where(a == -jnp.inf, 0.0, a)
    h_ref[...] = _DOT(a, wl_ref[...]) + _DOT(x_ref[...], wr_ref[...]) + b_ref[...]


def _bn_relu_body(h_ref, sum_ref, sq_ref, g_ref, be_ref, o_ref):
    mu = sum_ref[...] / NN
    var = sq_ref[...] / NN - mu * mu
    scale = g_ref[...] * lax.rsqrt(var + 1e-5)
    shift = be_ref[...] - mu * scale
    o_ref[...] = jnp.maximum(h_ref[...] * scale + shift, 0.0)


_ROWBLK = pl.BlockSpec((BN, DD), lambda i: (i, 0))
_FULLW = pl.BlockSpec((DD, DD), lambda i: (0, 0))
_VEC = pl.BlockSpec((1, DD), lambda i: (0, 0))


def _sage_stats(agg, x, wl, wr, b):
    return pl.pallas_call(
        _sage_stats_body,
        grid=(GRID,),
        in_specs=[_ROWBLK, _ROWBLK, _FULLW, _FULLW, _VEC],
        out_specs=[_ROWBLK, _VEC, _VEC],
        out_shape=[
            jax.ShapeDtypeStruct((NN, DD), jnp.float32),
            jax.ShapeDtypeStruct((1, DD), jnp.float32),
            jax.ShapeDtypeStruct((1, DD), jnp.float32),
        ],
    )(agg, x, wl, wr, b.reshape(1, DD))


def _sage(agg, x, wl, wr, b):
    return pl.pallas_call(
        _sage_body,
        grid=(GRID,),
        in_specs=[_ROWBLK, _ROWBLK, _FULLW, _FULLW, _VEC],
        out_specs=_ROWBLK,
        out_shape=jax.ShapeDtypeStruct((NN, DD), jnp.float32),
    )(agg, x, wl, wr, b.reshape(1, DD))


def _bn_relu(h, s, sq, gamma, beta):
    return pl.pallas_call(
        _bn_relu_body,
        grid=(GRID,),
        in_specs=[_ROWBLK, _VEC, _VEC, _VEC, _VEC],
        out_specs=_ROWBLK,
        out_shape=jax.ShapeDtypeStruct((NN, DD), jnp.float32),
    )(h, s, sq, gamma.reshape(1, DD), beta.reshape(1, DD))


def kernel(x, edge_index, W_l0, b_l0, W_r0, gamma0, beta0, W_l1, b_l1, W_r1):
    src = edge_index[0]
    dst = edge_index[1]
    slist, dlist, counts = _p1(src, dst)
    agg0 = _p2(x, slist, dlist, counts)[:NN]
    h_pre, s, sq = _sage_stats(agg0, x, W_l0, W_r0, b_l0)
    h = _bn_relu(h_pre, s, sq, gamma0, beta0)
    agg1 = _p2(h, slist, dlist, counts)[:NN]
    return _sage(agg1, h, W_l1, W_r1, b_l1)


# speedup vs baseline: 3.0666x; 1.2307x over previous
import functools

import jax
import jax.numpy as jnp
from jax import lax
from jax.experimental import pallas as pl
from jax.experimental.pallas import tpu as pltpu
from jax.experimental.pallas import tpu_sc as plsc

NN = 10000
EE = 320000
DD = 128

NTILE = 32
RPT = 320
ROWS = RPT + 1
NPAD = NTILE * RPT

SCH = 2000
NCHUNK = EE // SCH
GRP = SCH // 16

FLUSH = 2048
ABUF = 6144
CAP = FLUSH * 160
PADM = 256

LCH = 2048
GE = 64

_SC_PARAMS = pltpu.CompilerParams(needs_layout_passes=False)


def _wid():
    return lax.axis_index("s") * 2 + lax.axis_index("c")


def _p1_body(src_hbm, dst_hbm, slist, dlist, counts,
             sbuf0, sbuf1, dbuf0, dbuf1, asrc, adst, cbuf, ss0, ss1, sd0, sd1):
    wid = _wid()
    lo = wid * RPT
    sbufs = (sbuf0, sbuf1)
    dbufs = (dbuf0, dbuf1)
    ssems = (ss0, ss1)
    dsems = (sd0, sd1)

    for b in (0, 1):
        pltpu.async_copy(src_hbm.at[pl.ds(b * SCH, SCH)], sbufs[b], ssems[b])
        pltpu.async_copy(dst_hbm.at[pl.ds(b * SCH, SCH)], dbufs[b], dsems[b])

    def outer(p, carry):
        aoff, hoff = carry
        for b in (0, 1):
            c = p * 2 + b
            pltpu.make_async_copy(
                src_hbm.at[pl.ds(0, SCH)], sbufs[b], ssems[b]).wait()
            pltpu.make_async_copy(
                dst_hbm.at[pl.ds(0, SCH)], dbufs[b], dsems[b]).wait()

            for g in range(GRP):
                dvec = dbufs[b][pl.ds(g * 16, 16)]
                svec = sbufs[b][pl.ds(g * 16, 16)]
                dloc = dvec - lo
                mask = dloc.astype(jnp.uint32) < jnp.uint32(RPT)
                plsc.store_compressed(adst.at[pl.ds(aoff, 16)], dloc, mask=mask)
                plsc.store_compressed(asrc.at[pl.ds(aoff, 16)], svec, mask=mask)
                cntv = plsc.all_reduce_population_count(mask)
                cnt = cntv[0] if getattr(cntv, "ndim", 0) else cntv
                aoff = aoff + cnt

            do_flush = aoff >= FLUSH

            @pl.when(do_flush)
            def _(hoff=hoff):
                off = pl.multiple_of(wid * CAP + hoff, FLUSH)
                pltpu.sync_copy(asrc.at[pl.ds(0, FLUSH)],
                                slist.at[pl.ds(off, FLUSH)])
                pltpu.sync_copy(adst.at[pl.ds(0, FLUSH)],
                                dlist.at[pl.ds(off, FLUSH)])
                def mover(i, _):
                    asrc[pl.ds(i * 16, 16)] = asrc[pl.ds(FLUSH + i * 16, 16)]
                    adst[pl.ds(i * 16, 16)] = adst[pl.ds(FLUSH + i * 16, 16)]
                    return 0

                lax.fori_loop(0, (SCH + 15) >> 4, mover, 0)

            aoff = jnp.where(do_flush, aoff - FLUSH, aoff)
            hoff = jnp.where(do_flush, hoff + FLUSH, hoff)

            @pl.when(c + 2 < NCHUNK)
            def _(_b=b, _c=c):
                off = (_c + 2) * SCH
                pltpu.async_copy(src_hbm.at[pl.ds(off, SCH)], sbufs[_b],
                                 ssems[_b])
                pltpu.async_copy(dst_hbm.at[pl.ds(off, SCH)], dbufs[_b],
                                 dsems[_b])
        return aoff, hoff

    aoff, hoff = lax.fori_loop(0, NCHUNK // 2, outer, (0, 0))

    npad = (PADM - ((hoff + aoff) & (PADM - 1))) & (PADM - 1)
    padsrc = wid * RPT + lax.iota(jnp.int32, 16)
    padloc = jnp.full((16,), RPT, jnp.int32)

    def padloop(i, _):
        asrc[pl.ds(aoff + i * 16, 16)] = padsrc
        adst[pl.ds(aoff + i * 16, 16)] = padloc
        return 0

    lax.fori_loop(0, (npad + 15) >> 4, padloop, 0)
    aoff = aoff + npad

    for blk in (0, 1, 2):
        off = pl.multiple_of(wid * CAP + hoff + blk * FLUSH, FLUSH)
        pltpu.sync_copy(asrc.at[pl.ds(blk * FLUSH, FLUSH)],
                        slist.at[pl.ds(off, FLUSH)])
        pltpu.sync_copy(adst.at[pl.ds(blk * FLUSH, FLUSH)],
                        dlist.at[pl.ds(off, FLUSH)])

    total = hoff + aoff
    cbuf[pl.ds(0, 16)] = jnp.full((16,), 0, jnp.int32) + total
    pltpu.sync_copy(cbuf, counts.at[pl.ds(wid * 16, 16)])


def _p1(src, dst):
    mesh = plsc.VectorSubcoreMesh(core_axis_name="c", subcore_axis_name="s")
    fn = pl.kernel(
        _p1_body,
        out_type=(
            jax.ShapeDtypeStruct((NTILE * CAP,), jnp.int32),
            jax.ShapeDtypeStruct((NTILE * CAP,), jnp.int32),
            jax.ShapeDtypeStruct((NTILE * 16,), jnp.int32),
        ),
        mesh=mesh,
        compiler_params=_SC_PARAMS,
        scratch_types=[
            pltpu.VMEM((SCH,), jnp.int32),
            pltpu.VMEM((SCH,), jnp.int32),
            pltpu.VMEM((SCH,), jnp.int32),
            pltpu.VMEM((SCH,), jnp.int32),
            pltpu.VMEM((ABUF,), jnp.int32),
            pltpu.VMEM((ABUF,), jnp.int32),
            pltpu.VMEM((16,), jnp.int32),
            pltpu.SemaphoreType.DMA,
            pltpu.SemaphoreType.DMA,
            pltpu.SemaphoreType.DMA,
            pltpu.SemaphoreType.DMA,
        ],
    )
    return fn(src, dst)


def _p2_body(tab_hbm, slist, dlist, counts, out_hbm,
             acc, sbuf, dbuf, cbuf, msg0, msg1, msg2, msg3, m0, m1, m2, m3):
    wid = _wid()
    msgs = (msg0, msg1, msg2, msg3)
    msems = (m0, m1, m2, m3)

    pltpu.sync_copy(counts.at[pl.ds(wid * 16, 16)], cbuf)
    total = cbuf[pl.ds(0, 16)][0]

    neg = jnp.full((16,), -jnp.inf, jnp.float32)

    def initloop(r, _):
        for ci in range(8):
            acc[r, pl.ds(ci * 16, 16)] = neg
        return 0

    lax.fori_loop(0, ROWS, initloop, 0)

    nch = (total + (LCH - 1)) >> 11

    def chunk_body(ci, _):
        base = pl.multiple_of(wid * CAP + ci * LCH, LCH)
        pltpu.sync_copy(slist.at[pl.ds(base, LCH)], sbuf)
        pltpu.sync_copy(dlist.at[pl.ds(base, LCH)], dbuf)
        gcount = jnp.minimum(total - ci * LCH, LCH) >> 6

        for b in range(4):
            pltpu.async_copy(tab_hbm.at[sbuf.at[pl.ds(b * GE, GE)]],
                             msgs[b], msems[b])

        def quad_body(qd, _):
            for b in range(4):
                g = qd * 4 + b
                pltpu.make_async_copy(tab_hbm.at[pl.ds(0, GE)], msgs[b],
                                      msems[b]).wait()

                def sub(q, _, _b=b):
                    dvec = dbuf[pl.ds(g * GE + q * 16, 16)]
                    for j in range(16):
                        d = dvec[j]
                        e = q * 16 + j
                        for cc in range(8):
                            cur = acc[d, pl.ds(cc * 16, 16)]
                            m = msgs[_b][e, pl.ds(cc * 16, 16)]
                            acc[d, pl.ds(cc * 16, 16)] = jnp.maximum(cur, m)
                    return 0

                lax.fori_loop(0, 4, sub, 0)
                gg = jnp.minimum(g + 4, gcount - 1)
                pltpu.async_copy(tab_hbm.at[sbuf.at[pl.ds(gg * GE, GE)]],
                                 msgs[b], msems[b])
            return 0

        lax.fori_loop(0, gcount >> 2, quad_body, 0)
        for b in range(4):
            pltpu.make_async_copy(tab_hbm.at[pl.ds(0, GE)], msgs[b],
                                  msems[b]).wait()
        return 0

    lax.fori_loop(0, nch, chunk_body, 0)

    row0 = pl.multiple_of(wid * RPT, 8)
    pltpu.sync_copy(acc.at[pl.ds(0, RPT)], out_hbm.at[pl.ds(row0, RPT)])


def _p2(tab, slist, dlist, counts):
    mesh = plsc.VectorSubcoreMesh(core_axis_name="c", subcore_axis_name="s")
    fn = pl.kernel(
        _p2_body,
        out_type=jax.ShapeDtypeStruct((NPAD, DD), jnp.float32),
        mesh=mesh,
        compiler_params=_SC_PARAMS,
        scratch_types=[
            pltpu.VMEM((ROWS, DD), jnp.float32),
            pltpu.VMEM((LCH,), jnp.int32),
            pltpu.VMEM((LCH,), jnp.int32),
            pltpu.VMEM((16,), jnp.int32),
            pltpu.VMEM((GE, DD), jnp.float32),
            pltpu.VMEM((GE, DD), jnp.float32),
            pltpu.VMEM((GE, DD), jnp.float32),
            pltpu.VMEM((GE, DD), jnp.float32),
            pltpu.SemaphoreType.DMA,
            pltpu.SemaphoreType.DMA,
            pltpu.SemaphoreType.DMA,
            pltpu.SemaphoreType.DMA,
        ],
    )
    return fn(tab, slist, dlist, counts)


BN = 1000
GRID = NN // BN

_DOT = functools.partial(
    lax.dot_general,
    dimension_numbers=(((1,), (0,)), ((), ())),
    preferred_element_type=jnp.float32,
    precision=lax.Precision.HIGHEST,
)


def _sage_stats_body(a_ref, x_ref, wl_ref, wr_ref, b_ref,
                     h_ref, sum_ref, sq_ref):
    i = pl.program_id(0)
    a = a_ref[...]
    a = jnp.where(a == -jnp.inf, 0.0, a)
    h = _DOT(a, wl_ref[...]) + _DOT(x_ref[...], wr_ref[...]) + b_ref[...]
    h_ref[...] = h

    @pl.when(i == 0)
    def _():
        sum_ref[...] = jnp.zeros_like(sum_ref)
        sq_ref[...] = jnp.zeros_like(sq_ref)

    sum_ref[...] += jnp.sum(h, axis=0, keepdims=True)
    sq_ref[...] += jnp.sum(h * h, axis=0, keepdims=True)


def _sage_body(a_ref, x_ref, wl_ref, wr_ref, b_ref, h_ref):
    a = a_ref[...]
    a = jnp.where(a == -jnp.inf, 0.0, a)
    h_ref[...] = _DOT(a, wl_ref[...]) + _DOT(x_ref[...], wr_ref[...]) + b_ref[...]


def _bn_relu_body(h_ref, sum_ref, sq_ref, g_ref, be_ref, o_ref):
    mu = sum_ref[...] / NN
    var = sq_ref[...] / NN - mu * mu
    scale = g_ref[...] * lax.rsqrt(var + 1e-5)
    shift = be_ref[...] - mu * scale
    o_ref[...] = jnp.maximum(h_ref[...] * scale + shift, 0.0)


_ROWBLK = pl.BlockSpec((BN, DD), lambda i: (i, 0))
_FULLW = pl.BlockSpec((DD, DD), lambda i: (0, 0))
_VEC = pl.BlockSpec((1, DD), lambda i: (0, 0))


def _sage_stats(agg, x, wl, wr, b):
    return pl.pallas_call(
        _sage_stats_body,
        grid=(GRID,),
        in_specs=[_ROWBLK, _ROWBLK, _FULLW, _FULLW, _VEC],
        out_specs=[_ROWBLK, _VEC, _VEC],
        out_shape=[
            jax.ShapeDtypeStruct((NN, DD), jnp.float32),
            jax.ShapeDtypeStruct((1, DD), jnp.float32),
            jax.ShapeDtypeStruct((1, DD), jnp.float32),
        ],
    )(agg, x, wl, wr, b.reshape(1, DD))


def _sage(agg, x, wl, wr, b):
    return pl.pallas_call(
        _sage_body,
        grid=(GRID,),
        in_specs=[_ROWBLK, _ROWBLK, _FULLW, _FULLW, _VEC],
        out_specs=_ROWBLK,
        out_shape=jax.ShapeDtypeStruct((NN, DD), jnp.float32),
    )(agg, x, wl, wr, b.reshape(1, DD))


def _bn_relu(h, s, sq, gamma, beta):
    return pl.pallas_call(
        _bn_relu_body,
        grid=(GRID,),
        in_specs=[_ROWBLK, _VEC, _VEC, _VEC, _VEC],
        out_specs=_ROWBLK,
        out_shape=jax.ShapeDtypeStruct((NN, DD), jnp.float32),
    )(h, s, sq, gamma.reshape(1, DD), beta.reshape(1, DD))


def kernel(x, edge_index, W_l0, b_l0, W_r0, gamma0, beta0, W_l1, b_l1, W_r1):
    src = edge_index[0]
    dst = edge_index[1]
    slist, dlist, counts = _p1(src, dst)
    agg0 = _p2(x, slist, dlist, counts)[:NN]
    h_pre, s, sq = _sage_stats(agg0, x, W_l0, W_r0, b_l0)
    h = _bn_relu(h_pre, s, sq, gamma0, beta0)
    agg1 = _p2(h, slist, dlist, counts)[:NN]
    return _sage(agg1, h, W_l1, W_r1, b_l1)
